# Initial kernel scaffold; baseline (speedup 1.0000x reference)
import functools

import jax
import jax.numpy as jnp
from jax import lax
from jax.experimental import pallas as pl
from jax.experimental.pallas import tpu as pltpu
from jax.experimental.pallas import tpu_sc as plsc

_NW = 32
_B = 80


def _make_segsum(W, NPAD, E):
    EPW = E // _NW
    NCH = EPW // _B
    RPT = NPAD // 16
    RZ = RPT // _B
    assert E % _NW == 0 and EPW % _B == 0 and RPT % _B == 0
    WC = W // 16
    assert W % 16 == 0 and (W * 4) % 64 == 0

    mesh = plsc.VectorSubcoreMesh(core_axis_name="c", subcore_axis_name="s")

    @functools.partial(
        pl.kernel,
        mesh=mesh,
        out_type=jax.ShapeDtypeStruct((2 * NPAD, W), jnp.float32),
        scratch_types=[
            pltpu.VMEM((_B,), jnp.int32),
            pltpu.VMEM((_B,), jnp.int32),
            pltpu.VMEM((_B, W), jnp.float32),
            pltpu.VMEM_SHARED((NPAD, W), jnp.float32),
            pltpu.SemaphoreType.DMA,
        ],
    )
    def seg(table_hbm, src_hbm, dst_hbm, out_hbm, src_v, dst_v, rows_v, acc_sh, sem):
        cid = lax.axis_index("c")
        sid = lax.axis_index("s")
        zero16 = jnp.zeros((16,), jnp.float32)

        def zrow(r, carry):
            for j in range(WC):
                rows_v[r, pl.ds(j * 16, 16)] = zero16
            return carry

        lax.fori_loop(0, _B, zrow, 0)

        def zacc(k, carry):
            pltpu.sync_copy(rows_v, acc_sh.at[pl.ds(sid * RPT + k * _B, _B)])
            return carry

        lax.fori_loop(0, RZ, zacc, 0)
        plsc.subcore_barrier()

        base = (cid * 16 + sid) * EPW

        def step(g, carry):
            off = base + g * _B
            pltpu.sync_copy(src_hbm.at[pl.ds(off, _B)], src_v)
            pltpu.sync_copy(dst_hbm.at[pl.ds(off, _B)], dst_v)
            pltpu.async_copy(table_hbm.at[src_v], rows_v, sem).wait()
            pltpu.sync_copy(rows_v, acc_sh.at[dst_v], add=True)
            return carry

        lax.fori_loop(0, NCH, step, 0)
        plsc.subcore_barrier()

        pltpu.sync_copy(
            acc_sh.at[pl.ds(sid * RPT, RPT)],
            out_hbm.at[pl.ds(cid * NPAD + sid * RPT, RPT)],
        )

    return seg


def _tc1(xp, w1p, degp, NPAD):
    def body(x_ref, w_ref, d_ref, o_ref):
        deg = d_ref[0:NPAD, 0:1] + d_ref[NPAD : 2 * NPAD, 0:1] + 1.0
        dinv = lax.rsqrt(deg)
        h = jnp.dot(x_ref[...], w_ref[...],
                    preferred_element_type=jnp.float32,
                    precision=lax.Precision.HIGHEST)
        o_ref[...] = h * dinv

    return pl.pallas_call(
        body, out_shape=jax.ShapeDtypeStruct((NPAD, 112), jnp.float32)
    )(xp, w1p, degp)


def _tc2(u1, hp, degp, b1p, w2p, NPAD):
    def body(u_ref, h_ref, d_ref, b_ref, w_ref, o_ref):
        deg = d_ref[0:NPAD, 0:1] + d_ref[NPAD : 2 * NPAD, 0:1] + 1.0
        dinv = lax.rsqrt(deg)
        s = u_ref[0:NPAD, :] + u_ref[NPAD : 2 * NPAD, :] + h_ref[...]
        r = jnp.maximum(dinv * s + b_ref[...], 0.0)
        h2 = jnp.dot(r, w_ref[...],
                     preferred_element_type=jnp.float32,
                     precision=lax.Precision.HIGHEST)
        o_ref[...] = h2 * dinv

    return pl.pallas_call(
        body, out_shape=jax.ShapeDtypeStruct((NPAD, 16), jnp.float32)
    )(u1, hp, degp, b1p, w2p)


def _tc3(u2, t2, degp, b2p, NPAD):
    def body(u_ref, t_ref, d_ref, b_ref, o_ref):
        deg = d_ref[0:NPAD, 0:1] + d_ref[NPAD : 2 * NPAD, 0:1] + 1.0
        dinv = lax.rsqrt(deg)
        s = u_ref[0:NPAD, :] + u_ref[NPAD : 2 * NPAD, :] + t_ref[...]
        o_ref[...] = dinv * s + b_ref[...]

    return pl.pallas_call(
        body, out_shape=jax.ShapeDtypeStruct((NPAD, 16), jnp.float32)
    )(u2, t2, degp, b2p)


def kernel(x, edge_index, W1, b1, W2, b2):
    N, D = x.shape
    E = edge_index.shape[1]
    H = W1.shape[1]
    NPAD = -(-N // 512) * 512

    src = edge_index[0]
    dst = edge_index[1]
    xp = jnp.pad(x, ((0, NPAD - N), (0, 0)))
    w1p = jnp.pad(W1, ((0, 0), (0, 112 - H)))
    b1p = jnp.pad(b1, (0, 112 - H)).reshape(1, 112)
    w2p = jnp.pad(W2, ((0, 112 - H), (0, 15)))
    b2p = jnp.broadcast_to(b2.reshape(1, 1), (1, 16))
    ones_t = jnp.zeros((NPAD, 16), jnp.float32).at[:, 0].set(1.0)

    seg16 = _make_segsum(16, NPAD, E)
    seg112 = _make_segsum(112, NPAD, E)

    degp = seg16(ones_t, src, dst)
    hp = _tc1(xp, w1p, degp, NPAD)
    u1 = seg112(hp, src, dst)
    t2 = _tc2(u1, hp, degp, b1p, w2p, NPAD)
    u2 = seg16(t2, src, dst)
    o16 = _tc3(u2, t2, degp, b2p, NPAD)
    return o16[:N, 0:1]


# speedup vs baseline: 12.8779x; 12.8779x over previous
import functools

import jax
import jax.numpy as jnp
from jax import lax
from jax.experimental import pallas as pl
from jax.experimental.pallas import tpu as pltpu
from jax.experimental.pallas import tpu_sc as plsc

_NW = 32
_B = 80


def _make_segsum(W, NPAD, E):
    EPW = E // _NW
    NCH = EPW // _B
    RPT = NPAD // 16
    RZ = RPT // _B
    assert E % _NW == 0 and EPW % _B == 0 and RPT % _B == 0
    WC = W // 16
    assert W % 16 == 0 and (W * 4) % 64 == 0

    mesh = plsc.VectorSubcoreMesh(core_axis_name="c", subcore_axis_name="s")

    @functools.partial(
        pl.kernel,
        mesh=mesh,
        compiler_params=pltpu.CompilerParams(use_tc_tiling_on_sc=False),
        out_type=jax.ShapeDtypeStruct((2 * NPAD, W), jnp.float32),
        scratch_types=[
            pltpu.VMEM((_B,), jnp.int32),
            pltpu.VMEM((_B,), jnp.int32),
            pltpu.VMEM((_B, W), jnp.float32),
            pltpu.VMEM_SHARED((NPAD, W), jnp.float32),
            pltpu.SemaphoreType.DMA,
        ],
    )
    def seg(table_hbm, src_hbm, dst_hbm, out_hbm, src_v, dst_v, rows_v, acc_sh, sem):
        cid = lax.axis_index("c")
        sid = lax.axis_index("s")
        zero16 = jnp.zeros((16,), jnp.float32)

        def zrow(r, carry):
            for j in range(WC):
                rows_v[r, pl.ds(j * 16, 16)] = zero16
            return carry

        lax.fori_loop(0, _B, zrow, 0)

        def zacc(k, carry):
            pltpu.sync_copy(rows_v, acc_sh.at[pl.ds(sid * RPT + k * _B, _B)])
            return carry

        lax.fori_loop(0, RZ, zacc, 0)
        plsc.subcore_barrier()

        base = (cid * 16 + sid) * EPW

        def step(g, carry):
            off = base + g * _B
            pltpu.sync_copy(src_hbm.at[pl.ds(off, _B)], src_v)
            pltpu.sync_copy(dst_hbm.at[pl.ds(off, _B)], dst_v)
            pltpu.async_copy(table_hbm.at[src_v], rows_v, sem).wait()
            pltpu.sync_copy(rows_v, acc_sh.at[dst_v], add=True)
            return carry

        lax.fori_loop(0, NCH, step, 0)
        plsc.subcore_barrier()

        pltpu.sync_copy(
            acc_sh.at[pl.ds(sid * RPT, RPT)],
            out_hbm.at[pl.ds(cid * NPAD + sid * RPT, RPT)],
        )

    return seg


def _tc1(xp, w1p, degp, NPAD):
    def body(x_ref, w_ref, d_ref, o_ref):
        deg = d_ref[0:NPAD, 0:1] + d_ref[NPAD : 2 * NPAD, 0:1] + 1.0
        dinv = lax.rsqrt(deg)
        h = jnp.dot(x_ref[...], w_ref[...],
                    preferred_element_type=jnp.float32,
                    precision=lax.Precision.HIGHEST)
        o_ref[...] = h * dinv

    return pl.pallas_call(
        body, out_shape=jax.ShapeDtypeStruct((NPAD, 112), jnp.float32)
    )(xp, w1p, degp)


def _tc2(u1, hp, degp, b1p, w2p, NPAD):
    def body(u_ref, h_ref, d_ref, b_ref, w_ref, o_ref):
        deg = d_ref[0:NPAD, 0:1] + d_ref[NPAD : 2 * NPAD, 0:1] + 1.0
        dinv = lax.rsqrt(deg)
        s = u_ref[0:NPAD, :] + u_ref[NPAD : 2 * NPAD, :] + h_ref[...]
        r = jnp.maximum(dinv * s + b_ref[...], 0.0)
        h2 = jnp.dot(r, w_ref[...],
                     preferred_element_type=jnp.float32,
                     precision=lax.Precision.HIGHEST)
        o_ref[...] = h2 * dinv

    return pl.pallas_call(
        body, out_shape=jax.ShapeDtypeStruct((NPAD, 16), jnp.float32)
    )(u1, hp, degp, b1p, w2p)


def _tc3(u2, t2, degp, b2p, NPAD):
    def body(u_ref, t_ref, d_ref, b_ref, o_ref):
        deg = d_ref[0:NPAD, 0:1] + d_ref[NPAD : 2 * NPAD, 0:1] + 1.0
        dinv = lax.rsqrt(deg)
        s = u_ref[0:NPAD, :] + u_ref[NPAD : 2 * NPAD, :] + t_ref[...]
        o_ref[...] = dinv * s + b_ref[...]

    return pl.pallas_call(
        body, out_shape=jax.ShapeDtypeStruct((NPAD, 16), jnp.float32)
    )(u2, t2, degp, b2p)


def kernel(x, edge_index, W1, b1, W2, b2):
    N, D = x.shape
    E = edge_index.shape[1]
    H = W1.shape[1]
    NPAD = -(-N // 512) * 512

    src = edge_index[0]
    dst = edge_index[1]
    xp = jnp.pad(x, ((0, NPAD - N), (0, 0)))
    w1p = jnp.pad(W1, ((0, 0), (0, 112 - H)))
    b1p = jnp.pad(b1, (0, 112 - H)).reshape(1, 112)
    w2p = jnp.pad(W2, ((0, 112 - H), (0, 15)))
    b2p = jnp.broadcast_to(b2.reshape(1, 1), (1, 16))
    ones_t = jnp.zeros((NPAD, 16), jnp.float32).at[:, 0].set(1.0)

    seg16 = _make_segsum(16, NPAD, E)
    seg112 = _make_segsum(112, NPAD, E)

    degp = seg16(ones_t, src, dst)
    hp = _tc1(xp, w1p, degp, NPAD)
    u1 = seg112(hp, src, dst)
    t2 = _tc2(u1, hp, degp, b1p, w2p, NPAD)
    u2 = seg16(t2, src, dst)
    o16 = _tc3(u2, t2, degp, b2p, NPAD)
    return o16[:N, 0:1]


# speedup vs baseline: 39.4069x; 3.0600x over previous
import functools

import jax
import jax.numpy as jnp
from jax import lax
from jax.experimental import pallas as pl
from jax.experimental.pallas import tpu as pltpu
from jax.experimental.pallas import tpu_sc as plsc

_NW = 32
_B = 40
_NS = 5


def _zero_rows(rows_ref, s, nrow, wc):
    zero16 = jnp.zeros((16,), jnp.float32)

    def zrow(r, carry):
        for j in range(wc):
            rows_ref[s, r, pl.ds(j * 16, 16)] = zero16
        return carry

    lax.fori_loop(0, nrow, zrow, 0)


def _zero_acc(rows_ref, acc_sh, sid, rpt, sem):
    nz = rpt // _B
    cps = [
        pltpu.make_async_copy(
            rows_ref.at[0], acc_sh.at[pl.ds(sid * rpt + k * _B, _B)], sem
        )
        for k in range(nz)
    ]
    for cp in cps:
        cp.start()
    for cp in cps:
        cp.wait()


def _make_segsum(W, NPAD, E):
    EPW = E // _NW
    NCH = EPW // _B
    NH = NCH // _NS
    RPT = NPAD // 16
    assert E % _NW == 0 and EPW % _B == 0 and NCH % _NS == 0 and RPT % _B == 0
    WC = W // 16
    assert W % 16 == 0 and (W * 4) % 64 == 0

    mesh = plsc.VectorSubcoreMesh(core_axis_name="c", subcore_axis_name="s")

    @functools.partial(
        pl.kernel,
        mesh=mesh,
        compiler_params=pltpu.CompilerParams(use_tc_tiling_on_sc=False),
        out_type=jax.ShapeDtypeStruct((2 * NPAD, W), jnp.float32),
        scratch_types=[
            pltpu.VMEM((NCH, _B), jnp.int32),
            pltpu.VMEM((NCH, _B), jnp.int32),
            pltpu.VMEM((_NS, _B, W), jnp.float32),
            pltpu.VMEM_SHARED((NPAD, W), jnp.float32),
            pltpu.SemaphoreType.DMA((_NS,)),
            pltpu.SemaphoreType.DMA((_NS,)),
            pltpu.SemaphoreType.DMA,
        ],
    )
    def seg(table_hbm, src_hbm, dst_hbm, out_hbm, src_l, dst_l, rows_v,
            acc_sh, gsem, ssem, zsem):
        cid = lax.axis_index("c")
        sid = lax.axis_index("s")
        wid = cid * 16 + sid

        pltpu.make_async_copy(src_hbm.at[pl.ds(wid * NCH, NCH)], src_l, zsem).start()
        pltpu.make_async_copy(dst_hbm.at[pl.ds(wid * NCH, NCH)], dst_l, zsem).start()
        _zero_rows(rows_v, 0, _B, WC)
        pltpu.make_async_copy(src_hbm.at[pl.ds(wid * NCH, NCH)], src_l, zsem).wait()
        pltpu.make_async_copy(dst_hbm.at[pl.ds(wid * NCH, NCH)], dst_l, zsem).wait()
        _zero_acc(rows_v, acc_sh, sid, RPT, zsem)
        plsc.subcore_barrier()

        def gather(g, s):
            return pltpu.make_async_copy(
                table_hbm.at[src_l.at[g]], rows_v.at[s], gsem.at[s]
            )

        def scatter(g, s):
            return pltpu.make_async_copy(
                rows_v.at[s], acc_sh.at[dst_l.at[g]], ssem.at[s]
            )

        for s in range(_NS):
            gather(s, s).start()

        def hyper(h, carry):
            g0 = h * _NS
            for s in range(_NS):
                gather(g0 + s, s).wait()
                scatter(g0 + s, s).start(add=True)
            for s in range(_NS):
                scatter(g0 + s, s).wait()
                gather(g0 + _NS + s, s).start()
            return carry

        lax.fori_loop(0, NH - 1, hyper, 0)

        g0 = (NH - 1) * _NS
        for s in range(_NS):
            gather(g0 + s, s).wait()
            scatter(g0 + s, s).start(add=True)
        for s in range(_NS):
            scatter(g0 + s, s).wait()

        plsc.subcore_barrier()
        pltpu.sync_copy(
            acc_sh.at[pl.ds(sid * RPT, RPT)],
            out_hbm.at[pl.ds(cid * NPAD + sid * RPT, RPT)],
        )

    return seg


def _make_deghist(NPAD, E):
    EPW = E // _NW
    NCH = EPW // _B
    RPT = NPAD // 16
    assert E % _NW == 0 and EPW % _B == 0 and RPT % _B == 0
    W = 16

    mesh = plsc.VectorSubcoreMesh(core_axis_name="c", subcore_axis_name="s")

    @functools.partial(
        pl.kernel,
        mesh=mesh,
        compiler_params=pltpu.CompilerParams(use_tc_tiling_on_sc=False),
        out_type=jax.ShapeDtypeStruct((2 * NPAD, W), jnp.float32),
        scratch_types=[
            pltpu.VMEM((NCH, _B), jnp.int32),
            pltpu.VMEM((_NS, _B, W), jnp.float32),
            pltpu.VMEM_SHARED((NPAD, W), jnp.float32),
            pltpu.SemaphoreType.DMA((_NS,)),
            pltpu.SemaphoreType.DMA,
        ],
    )
    def deg(dst_hbm, out_hbm, dst_l, rows_v, acc_sh, ssem, zsem):
        cid = lax.axis_index("c")
        sid = lax.axis_index("s")
        wid = cid * 16 + sid

        pltpu.make_async_copy(dst_hbm.at[pl.ds(wid * NCH, NCH)], dst_l, zsem).start()
        _zero_rows(rows_v, 0, _B, 1)
        one16 = jnp.where(lax.iota(jnp.int32, 16) == 0, 1.0, 0.0).astype(jnp.float32)

        def orow(r, carry):
            rows_v[0, r, pl.ds(0, 16)] = one16
            return carry

        pltpu.make_async_copy(dst_hbm.at[pl.ds(wid * NCH, NCH)], dst_l, zsem).wait()
        _zero_acc(rows_v, acc_sh, sid, RPT, zsem)
        lax.fori_loop(0, _B, orow, 0)
        plsc.subcore_barrier()

        def scatter(g, s):
            return pltpu.make_async_copy(
                rows_v.at[0], acc_sh.at[dst_l.at[g]], ssem.at[s]
            )

        for s in range(_NS):
            scatter(s, s).start(add=True)

        def hyper(h, carry):
            g0 = h * _NS
            for s in range(_NS):
                scatter(g0 + s, s).wait()
                scatter(g0 + _NS + s, s).start(add=True)
            return carry

        lax.fori_loop(0, NCH // _NS - 1, hyper, 0)
        g0 = NCH - _NS
        for s in range(_NS):
            scatter(g0 + s, s).wait()

        plsc.subcore_barrier()
        pltpu.sync_copy(
            acc_sh.at[pl.ds(sid * RPT, RPT)],
            out_hbm.at[pl.ds(cid * NPAD + sid * RPT, RPT)],
        )

    return deg


def _tc_matmul(xp, w1p, NPAD):
    def body(x_ref, w_ref, o_ref):
        o_ref[...] = jnp.dot(x_ref[...], w_ref[...],
                             preferred_element_type=jnp.float32,
                             precision=lax.Precision.HIGHEST)

    return pl.pallas_call(
        body, out_shape=jax.ShapeDtypeStruct((NPAD, 112), jnp.float32)
    )(xp, w1p)


def _tc_scale(h, degp, NPAD):
    def body(h_ref, d_ref, o_ref):
        deg = d_ref[0:NPAD, 0:1] + d_ref[NPAD : 2 * NPAD, 0:1] + 1.0
        o_ref[...] = h_ref[...] * lax.rsqrt(deg)

    return pl.pallas_call(
        body, out_shape=jax.ShapeDtypeStruct((NPAD, 112), jnp.float32)
    )(h, degp)


def _tc2(u1, hp, degp, b1p, w2p, NPAD):
    def body(u_ref, h_ref, d_ref, b_ref, w_ref, o_ref):
        deg = d_ref[0:NPAD, 0:1] + d_ref[NPAD : 2 * NPAD, 0:1] + 1.0
        dinv = lax.rsqrt(deg)
        s = u_ref[0:NPAD, :] + u_ref[NPAD : 2 * NPAD, :] + h_ref[...]
        r = jnp.maximum(dinv * s + b_ref[...], 0.0)
        h2 = jnp.dot(r, w_ref[...],
                     preferred_element_type=jnp.float32,
                     precision=lax.Precision.HIGHEST)
        o_ref[...] = h2 * dinv

    return pl.pallas_call(
        body, out_shape=jax.ShapeDtypeStruct((NPAD, 16), jnp.float32)
    )(u1, hp, degp, b1p, w2p)


def _tc3(u2, t2, degp, b2p, NPAD):
    def body(u_ref, t_ref, d_ref, b_ref, o_ref):
        deg = d_ref[0:NPAD, 0:1] + d_ref[NPAD : 2 * NPAD, 0:1] + 1.0
        dinv = lax.rsqrt(deg)
        s = u_ref[0:NPAD, :] + u_ref[NPAD : 2 * NPAD, :] + t_ref[...]
        o_ref[...] = dinv * s + b_ref[...]

    return pl.pallas_call(
        body, out_shape=jax.ShapeDtypeStruct((NPAD, 16), jnp.float32)
    )(u2, t2, degp, b2p)


def kernel(x, edge_index, W1, b1, W2, b2):
    N, D = x.shape
    E = edge_index.shape[1]
    H = W1.shape[1]
    NPAD = -(-N // 512) * 512

    src = edge_index[0].reshape(E // _B, _B)
    dst = edge_index[1].reshape(E // _B, _B)
    xp = jnp.pad(x, ((0, NPAD - N), (0, 0)))
    w1p = jnp.pad(W1, ((0, 0), (0, 112 - H)))
    b1p = jnp.pad(b1, (0, 112 - H)).reshape(1, 112)
    w2p = jnp.pad(W2, ((0, 112 - H), (0, 15)))
    b2p = jnp.broadcast_to(b2.reshape(1, 1), (1, 16))

    seg16 = _make_segsum(16, NPAD, E)
    seg112 = _make_segsum(112, NPAD, E)
    deghist = _make_deghist(NPAD, E)

    degp = deghist(dst)
    h = _tc_matmul(xp, w1p, NPAD)
    hp = _tc_scale(h, degp, NPAD)
    u1 = seg112(hp, src, dst)
    t2 = _tc2(u1, hp, degp, b1p, w2p, NPAD)
    u2 = seg16(t2, src, dst)
    o16 = _tc3(u2, t2, degp, b2p, NPAD)
    return o16[:N, 0:1]


# speedup vs baseline: 41.4186x; 1.0511x over previous
import functools

import jax
import jax.numpy as jnp
from jax import lax
from jax.experimental import pallas as pl
from jax.experimental.pallas import tpu as pltpu
from jax.experimental.pallas import tpu_sc as plsc

_NW = 32
_NS = 5


def _zero_rows(rows_ref, s, nrow, wc):
    zero16 = jnp.zeros((16,), jnp.float32)

    def zrow(r, carry):
        for j in range(wc):
            rows_ref[s, r, pl.ds(j * 16, 16)] = zero16
        return carry

    lax.fori_loop(0, nrow, zrow, 0)


def _zero_acc(rows_ref, acc_sh, sid, rpt, sem, B):
    nz = rpt // B
    cps = [
        pltpu.make_async_copy(
            rows_ref.at[0], acc_sh.at[pl.ds(sid * rpt + k * B, B)], sem
        )
        for k in range(nz)
    ]
    for cp in cps:
        cp.start()
    for cp in cps:
        cp.wait()


def _make_segsum(W, NPAD, E, B):
    EPW = E // _NW
    NCH = EPW // B
    NH = NCH // _NS
    RPT = NPAD // 16
    assert E % _NW == 0 and EPW % B == 0 and NCH % _NS == 0 and RPT % B == 0
    WC = W // 16
    assert W % 16 == 0 and (W * 4) % 64 == 0

    mesh = plsc.VectorSubcoreMesh(core_axis_name="c", subcore_axis_name="s")

    @functools.partial(
        pl.kernel,
        mesh=mesh,
        compiler_params=pltpu.CompilerParams(use_tc_tiling_on_sc=False),
        out_type=jax.ShapeDtypeStruct((2 * NPAD, W), jnp.float32),
        scratch_types=[
            pltpu.VMEM((NCH, B), jnp.int32),
            pltpu.VMEM((NCH, B), jnp.int32),
            pltpu.VMEM((_NS, B, W), jnp.float32),
            pltpu.VMEM_SHARED((NPAD, W), jnp.float32),
            pltpu.SemaphoreType.DMA((_NS,)),
            pltpu.SemaphoreType.DMA((_NS,)),
            pltpu.SemaphoreType.DMA,
        ],
    )
    def seg(table_hbm, src_hbm, dst_hbm, out_hbm, src_l, dst_l, rows_v,
            acc_sh, gsem, ssem, zsem):
        cid = lax.axis_index("c")
        sid = lax.axis_index("s")
        wid = cid * 16 + sid

        pltpu.make_async_copy(src_hbm.at[pl.ds(wid * NCH, NCH)], src_l, zsem).start()
        pltpu.make_async_copy(dst_hbm.at[pl.ds(wid * NCH, NCH)], dst_l, zsem).start()
        _zero_rows(rows_v, 0, B, WC)
        pltpu.make_async_copy(src_hbm.at[pl.ds(wid * NCH, NCH)], src_l, zsem).wait()
        pltpu.make_async_copy(dst_hbm.at[pl.ds(wid * NCH, NCH)], dst_l, zsem).wait()
        _zero_acc(rows_v, acc_sh, sid, RPT, zsem, B)
        plsc.subcore_barrier()

        def gather(g, s):
            return pltpu.make_async_copy(
                table_hbm.at[src_l.at[g]], rows_v.at[s], gsem.at[s]
            )

        def scatter(g, s):
            return pltpu.make_async_copy(
                rows_v.at[s], acc_sh.at[dst_l.at[g]], ssem.at[s]
            )

        for s in range(_NS):
            gather(s, s).start()

        def hyper(h, carry):
            g0 = h * _NS
            for s in range(_NS):
                gather(g0 + s, s).wait()
                scatter(g0 + s, s).start(add=True)
            for s in range(_NS):
                scatter(g0 + s, s).wait()
                gather(g0 + _NS + s, s).start()
            return carry

        lax.fori_loop(0, NH - 1, hyper, 0)

        g0 = (NH - 1) * _NS
        for s in range(_NS):
            gather(g0 + s, s).wait()
            scatter(g0 + s, s).start(add=True)
        for s in range(_NS):
            scatter(g0 + s, s).wait()

        plsc.subcore_barrier()
        pltpu.sync_copy(
            acc_sh.at[pl.ds(sid * RPT, RPT)],
            out_hbm.at[pl.ds(cid * NPAD + sid * RPT, RPT)],
        )

    return seg


def _make_deghist(NPAD, E, B):
    EPW = E // _NW
    NCH = EPW // B
    RPT = NPAD // 16
    assert E % _NW == 0 and EPW % B == 0 and RPT % B == 0
    W = 16

    mesh = plsc.VectorSubcoreMesh(core_axis_name="c", subcore_axis_name="s")

    @functools.partial(
        pl.kernel,
        mesh=mesh,
        compiler_params=pltpu.CompilerParams(use_tc_tiling_on_sc=False),
        out_type=jax.ShapeDtypeStruct((2 * NPAD, W), jnp.float32),
        scratch_types=[
            pltpu.VMEM((NCH, B), jnp.int32),
            pltpu.VMEM((_NS, B, W), jnp.float32),
            pltpu.VMEM_SHARED((NPAD, W), jnp.float32),
            pltpu.SemaphoreType.DMA((_NS,)),
            pltpu.SemaphoreType.DMA,
        ],
    )
    def deg(dst_hbm, out_hbm, dst_l, rows_v, acc_sh, ssem, zsem):
        cid = lax.axis_index("c")
        sid = lax.axis_index("s")
        wid = cid * 16 + sid

        pltpu.make_async_copy(dst_hbm.at[pl.ds(wid * NCH, NCH)], dst_l, zsem).start()
        _zero_rows(rows_v, 0, B, 1)
        one16 = jnp.where(lax.iota(jnp.int32, 16) == 0, 1.0, 0.0).astype(jnp.float32)

        def orow(r, carry):
            rows_v[0, r, pl.ds(0, 16)] = one16
            return carry

        pltpu.make_async_copy(dst_hbm.at[pl.ds(wid * NCH, NCH)], dst_l, zsem).wait()
        _zero_acc(rows_v, acc_sh, sid, RPT, zsem, B)
        lax.fori_loop(0, B, orow, 0)
        plsc.subcore_barrier()

        def scatter(g, s):
            return pltpu.make_async_copy(
                rows_v.at[0], acc_sh.at[dst_l.at[g]], ssem.at[s]
            )

        for s in range(_NS):
            scatter(s, s).start(add=True)

        def hyper(h, carry):
            g0 = h * _NS
            for s in range(_NS):
                scatter(g0 + s, s).wait()
                scatter(g0 + _NS + s, s).start(add=True)
            return carry

        lax.fori_loop(0, NCH // _NS - 1, hyper, 0)
        g0 = NCH - _NS
        for s in range(_NS):
            scatter(g0 + s, s).wait()

        plsc.subcore_barrier()
        pltpu.sync_copy(
            acc_sh.at[pl.ds(sid * RPT, RPT)],
            out_hbm.at[pl.ds(cid * NPAD + sid * RPT, RPT)],
        )

    return deg


def _tc1(xp, w1p, degp, NPAD):
    def body(x_ref, w_ref, d_ref, o_ref):
        deg = d_ref[0:NPAD, 0:1] + d_ref[NPAD : 2 * NPAD, 0:1] + 1.0
        h = jnp.dot(x_ref[...], w_ref[...],
                    preferred_element_type=jnp.float32,
                    precision=lax.Precision.HIGHEST)
        o_ref[...] = h * lax.rsqrt(deg)

    return pl.pallas_call(
        body, out_shape=jax.ShapeDtypeStruct((NPAD, 112), jnp.float32)
    )(xp, w1p, degp)


def _tc2(u1, hp, degp, b1p, w2p, NPAD):
    def body(u_ref, h_ref, d_ref, b_ref, w_ref, o_ref):
        deg = d_ref[0:NPAD, 0:1] + d_ref[NPAD : 2 * NPAD, 0:1] + 1.0
        dinv = lax.rsqrt(deg)
        s = u_ref[0:NPAD, :] + u_ref[NPAD : 2 * NPAD, :] + h_ref[...]
        r = jnp.maximum(dinv * s + b_ref[...], 0.0)
        h2 = jnp.dot(r, w_ref[...],
                     preferred_element_type=jnp.float32,
                     precision=lax.Precision.HIGHEST)
        o_ref[...] = h2 * dinv

    return pl.pallas_call(
        body, out_shape=jax.ShapeDtypeStruct((NPAD, 16), jnp.float32)
    )(u1, hp, degp, b1p, w2p)


def _tc3(u2, t2, degp, b2p, NPAD):
    def body(u_ref, t_ref, d_ref, b_ref, o_ref):
        deg = d_ref[0:NPAD, 0:1] + d_ref[NPAD : 2 * NPAD, 0:1] + 1.0
        dinv = lax.rsqrt(deg)
        s = u_ref[0:NPAD, :] + u_ref[NPAD : 2 * NPAD, :] + t_ref[...]
        o_ref[...] = dinv * s + b_ref[...]

    return pl.pallas_call(
        body, out_shape=jax.ShapeDtypeStruct((NPAD, 16), jnp.float32)
    )(u2, t2, degp, b2p)


def kernel(x, edge_index, W1, b1, W2, b2):
    N, D = x.shape
    E = edge_index.shape[1]
    H = W1.shape[1]
    NPAD = -(-N // 512) * 512

    B16, B112 = 80, 40
    src40 = edge_index[0].reshape(E // B112, B112)
    dst40 = edge_index[1].reshape(E // B112, B112)
    src80 = edge_index[0].reshape(E // B16, B16)
    dst80 = edge_index[1].reshape(E // B16, B16)
    xp = jnp.pad(x, ((0, NPAD - N), (0, 0)))
    w1p = jnp.pad(W1, ((0, 0), (0, 112 - H)))
    b1p = jnp.pad(b1, (0, 112 - H)).reshape(1, 112)
    w2p = jnp.pad(W2, ((0, 112 - H), (0, 15)))
    b2p = jnp.broadcast_to(b2.reshape(1, 1), (1, 16))

    seg16 = _make_segsum(16, NPAD, E, B16)
    seg112 = _make_segsum(112, NPAD, E, B112)
    deghist = _make_deghist(NPAD, E, B16)

    degp = deghist(dst80)
    hp = _tc1(xp, w1p, degp, NPAD)
    u1 = seg112(hp, src40, dst40)
    t2 = _tc2(u1, hp, degp, b1p, w2p, NPAD)
    u2 = seg16(t2, src80, dst80)
    o16 = _tc3(u2, t2, degp, b2p, NPAD)
    return o16[:N, 0:1]


# speedup vs baseline: 41.6647x; 1.0059x over previous
import functools

import jax
import jax.numpy as jnp
from jax import lax
from jax.experimental import pallas as pl
from jax.experimental.pallas import tpu as pltpu
from jax.experimental.pallas import tpu_sc as plsc

_NW = 32
_NS = 5


def _zero_rows(rows_ref, s, nrow, wc):
    zero16 = jnp.zeros((16,), jnp.float32)

    def zrow(r, carry):
        for j in range(wc):
            rows_ref[s, r, pl.ds(j * 16, 16)] = zero16
        return carry

    lax.fori_loop(0, nrow, zrow, 0)


def _zero_acc(rows_ref, acc_sh, sid, rpt, sem, B):
    nz = rpt // B
    cps = [
        pltpu.make_async_copy(
            rows_ref.at[0], acc_sh.at[pl.ds(sid * rpt + k * B, B)], sem
        )
        for k in range(nz)
    ]
    for cp in cps:
        cp.start()
    for cp in cps:
        cp.wait()


def _make_segsum(W, NPAD, E, B):
    EPW = E // _NW
    NCH = EPW // B
    NH = NCH // _NS
    RPT = NPAD // 16
    assert E % _NW == 0 and EPW % B == 0 and NCH % _NS == 0 and RPT % B == 0
    WC = W // 16
    assert W % 16 == 0 and (W * 4) % 64 == 0

    mesh = plsc.VectorSubcoreMesh(core_axis_name="c", subcore_axis_name="s")

    @functools.partial(
        pl.kernel,
        mesh=mesh,
        compiler_params=pltpu.CompilerParams(use_tc_tiling_on_sc=False),
        out_type=jax.ShapeDtypeStruct((2 * NPAD, W), jnp.float32),
        scratch_types=[
            pltpu.VMEM((NCH, B), jnp.int32),
            pltpu.VMEM((NCH, B), jnp.int32),
            pltpu.VMEM((_NS, B, W), jnp.float32),
            pltpu.VMEM_SHARED((NPAD, W), jnp.float32),
            pltpu.SemaphoreType.DMA((_NS,)),
            pltpu.SemaphoreType.DMA((_NS,)),
            pltpu.SemaphoreType.DMA,
        ],
    )
    def seg(table_hbm, src_hbm, dst_hbm, out_hbm, src_l, dst_l, rows_v,
            acc_sh, gsem, ssem, zsem):
        cid = lax.axis_index("c")
        sid = lax.axis_index("s")
        wid = cid * 16 + sid

        pltpu.make_async_copy(src_hbm.at[pl.ds(wid * NCH, NCH)], src_l, zsem).start()
        pltpu.make_async_copy(dst_hbm.at[pl.ds(wid * NCH, NCH)], dst_l, zsem).start()
        _zero_rows(rows_v, 0, B, WC)
        pltpu.make_async_copy(src_hbm.at[pl.ds(wid * NCH, NCH)], src_l, zsem).wait()
        pltpu.make_async_copy(dst_hbm.at[pl.ds(wid * NCH, NCH)], dst_l, zsem).wait()
        _zero_acc(rows_v, acc_sh, sid, RPT, zsem, B)
        plsc.subcore_barrier()

        def gather(g, s):
            return pltpu.make_async_copy(
                table_hbm.at[src_l.at[g]], rows_v.at[s], gsem.at[s]
            )

        def scatter(g, s):
            return pltpu.make_async_copy(
                rows_v.at[s], acc_sh.at[dst_l.at[g]], ssem.at[s]
            )

        for s in range(_NS):
            gather(s, s).start()

        def hyper(h, carry):
            g0 = h * _NS
            for s in range(_NS):
                gather(g0 + s, s).wait()
                scatter(g0 + s, s).start(add=True)
            for s in range(_NS):
                scatter(g0 + s, s).wait()
                gather(g0 + _NS + s, s).start()
            return carry

        lax.fori_loop(0, NH - 1, hyper, 0)

        g0 = (NH - 1) * _NS
        for s in range(_NS):
            gather(g0 + s, s).wait()
            scatter(g0 + s, s).start(add=True)
        for s in range(_NS):
            scatter(g0 + s, s).wait()

        plsc.subcore_barrier()
        pltpu.sync_copy(
            acc_sh.at[pl.ds(sid * RPT, RPT)],
            out_hbm.at[pl.ds(cid * NPAD + sid * RPT, RPT)],
        )

    return seg


def _make_segsum_sb(W, NPAD, E, B):
    EPW = E // _NW
    NCH = EPW // B
    SB = 5 * _NS
    NSB = NCH // SB
    RPT = NPAD // 16
    assert E % _NW == 0 and EPW % B == 0 and NCH % SB == 0 and RPT % B == 0
    WC = W // 16
    assert W % 16 == 0 and (W * 4) % 64 == 0

    mesh = plsc.VectorSubcoreMesh(core_axis_name="c", subcore_axis_name="s")

    @functools.partial(
        pl.kernel,
        mesh=mesh,
        compiler_params=pltpu.CompilerParams(use_tc_tiling_on_sc=False),
        out_type=jax.ShapeDtypeStruct((2 * NPAD, W), jnp.float32),
        scratch_types=[
            pltpu.VMEM((2 * SB, B), jnp.int32),
            pltpu.VMEM((2 * SB, B), jnp.int32),
            pltpu.VMEM((_NS, B, W), jnp.float32),
            pltpu.VMEM_SHARED((NPAD, W), jnp.float32),
            pltpu.SemaphoreType.DMA((_NS,)),
            pltpu.SemaphoreType.DMA((_NS,)),
            pltpu.SemaphoreType.DMA,
        ],
    )
    def seg(table_hbm, src_hbm, dst_hbm, out_hbm, src_l, dst_l, rows_v,
            acc_sh, gsem, ssem, zsem):
        cid = lax.axis_index("c")
        sid = lax.axis_index("s")
        wid = cid * 16 + sid
        base = wid * NCH

        def stage(sb, buf):
            return (
                pltpu.make_async_copy(
                    src_hbm.at[pl.ds(base + sb * SB, SB)],
                    src_l.at[pl.ds(buf * SB, SB)], zsem),
                pltpu.make_async_copy(
                    dst_hbm.at[pl.ds(base + sb * SB, SB)],
                    dst_l.at[pl.ds(buf * SB, SB)], zsem),
            )

        for cp in stage(0, 0):
            cp.start()
        _zero_rows(rows_v, 0, B, WC)
        for cp in stage(0, 0):
            cp.wait()
        _zero_acc(rows_v, acc_sh, sid, RPT, zsem, B)
        plsc.subcore_barrier()

        def gather(buf, q, s):
            return pltpu.make_async_copy(
                table_hbm.at[src_l.at[buf * SB + q]], rows_v.at[s], gsem.at[s]
            )

        def scatter(buf, q, s):
            return pltpu.make_async_copy(
                rows_v.at[s], acc_sh.at[dst_l.at[buf * SB + q]], ssem.at[s]
            )

        for s in range(_NS):
            gather(0, s, s).start()

        def sbloop(sb, carry):
            b = sb % 2
            last = sb == NSB - 1

            @pl.when(jnp.logical_not(last))
            def _():
                for cp in stage(sb + 1, 1 - b):
                    cp.start()

            def hyper(h, c2):
                q0 = h * _NS
                for s in range(_NS):
                    gather(b, q0 + s, s).wait()
                    scatter(b, q0 + s, s).start(add=True)
                for s in range(_NS):
                    scatter(b, q0 + s, s).wait()
                    gather(b, q0 + _NS + s, s).start()
                return c2

            lax.fori_loop(0, SB // _NS - 1, hyper, 0)
            q0 = SB - _NS
            for s in range(_NS):
                gather(b, q0 + s, s).wait()
                scatter(b, q0 + s, s).start(add=True)
            for s in range(_NS):
                scatter(b, q0 + s, s).wait()

            @pl.when(jnp.logical_not(last))
            def _():
                for cp in stage(sb + 1, 1 - b):
                    cp.wait()
                for s in range(_NS):
                    gather(1 - b, s, s).start()

            return carry

        lax.fori_loop(0, NSB, sbloop, 0)

        plsc.subcore_barrier()
        pltpu.sync_copy(
            acc_sh.at[pl.ds(sid * RPT, RPT)],
            out_hbm.at[pl.ds(cid * NPAD + sid * RPT, RPT)],
        )

    return seg


def _make_deghist(NPAD, E, B):
    EPW = E // _NW
    NCH = EPW // B
    RPT = NPAD // 16
    assert E % _NW == 0 and EPW % B == 0 and RPT % B == 0
    W = 16

    mesh = plsc.VectorSubcoreMesh(core_axis_name="c", subcore_axis_name="s")

    @functools.partial(
        pl.kernel,
        mesh=mesh,
        compiler_params=pltpu.CompilerParams(use_tc_tiling_on_sc=False),
        out_type=jax.ShapeDtypeStruct((2 * NPAD, W), jnp.float32),
        scratch_types=[
            pltpu.VMEM((NCH, B), jnp.int32),
            pltpu.VMEM((_NS, B, W), jnp.float32),
            pltpu.VMEM_SHARED((NPAD, W), jnp.float32),
            pltpu.SemaphoreType.DMA((_NS,)),
            pltpu.SemaphoreType.DMA,
        ],
    )
    def deg(dst_hbm, out_hbm, dst_l, rows_v, acc_sh, ssem, zsem):
        cid = lax.axis_index("c")
        sid = lax.axis_index("s")
        wid = cid * 16 + sid

        pltpu.make_async_copy(dst_hbm.at[pl.ds(wid * NCH, NCH)], dst_l, zsem).start()
        _zero_rows(rows_v, 0, B, 1)
        one16 = jnp.where(lax.iota(jnp.int32, 16) == 0, 1.0, 0.0).astype(jnp.float32)

        def orow(r, carry):
            rows_v[0, r, pl.ds(0, 16)] = one16
            return carry

        pltpu.make_async_copy(dst_hbm.at[pl.ds(wid * NCH, NCH)], dst_l, zsem).wait()
        _zero_acc(rows_v, acc_sh, sid, RPT, zsem, B)
        lax.fori_loop(0, B, orow, 0)
        plsc.subcore_barrier()

        def scatter(g, s):
            return pltpu.make_async_copy(
                rows_v.at[0], acc_sh.at[dst_l.at[g]], ssem.at[s]
            )

        for s in range(_NS):
            scatter(s, s).start(add=True)

        def hyper(h, carry):
            g0 = h * _NS
            for s in range(_NS):
                scatter(g0 + s, s).wait()
                scatter(g0 + _NS + s, s).start(add=True)
            return carry

        lax.fori_loop(0, NCH // _NS - 1, hyper, 0)
        g0 = NCH - _NS
        for s in range(_NS):
            scatter(g0 + s, s).wait()

        plsc.subcore_barrier()
        pltpu.sync_copy(
            acc_sh.at[pl.ds(sid * RPT, RPT)],
            out_hbm.at[pl.ds(cid * NPAD + sid * RPT, RPT)],
        )

    return deg


def _tc1(xp, w1p, degp, NPAD):
    def body(x_ref, w_ref, d_ref, o_ref):
        deg = d_ref[0:NPAD, 0:1] + d_ref[NPAD : 2 * NPAD, 0:1] + 1.0
        h = jnp.dot(x_ref[...], w_ref[...],
                    preferred_element_type=jnp.float32,
                    precision=lax.Precision.HIGHEST)
        o_ref[...] = h * lax.rsqrt(deg)

    return pl.pallas_call(
        body, out_shape=jax.ShapeDtypeStruct((NPAD, 112), jnp.float32)
    )(xp, w1p, degp)


def _tc2(u1, hp, degp, b1p, w2p, NPAD):
    def body(u_ref, h_ref, d_ref, b_ref, w_ref, o_ref):
        deg = d_ref[0:NPAD, 0:1] + d_ref[NPAD : 2 * NPAD, 0:1] + 1.0
        dinv = lax.rsqrt(deg)
        s = u_ref[0:NPAD, :] + u_ref[NPAD : 2 * NPAD, :] + h_ref[...]
        r = jnp.maximum(dinv * s + b_ref[...], 0.0)
        h2 = jnp.dot(r, w_ref[...],
                     preferred_element_type=jnp.float32,
                     precision=lax.Precision.HIGHEST)
        o_ref[...] = h2 * dinv

    return pl.pallas_call(
        body, out_shape=jax.ShapeDtypeStruct((NPAD, 16), jnp.float32)
    )(u1, hp, degp, b1p, w2p)


def _tc3(u2, t2, degp, b2p, NPAD):
    def body(u_ref, t_ref, d_ref, b_ref, o_ref):
        deg = d_ref[0:NPAD, 0:1] + d_ref[NPAD : 2 * NPAD, 0:1] + 1.0
        dinv = lax.rsqrt(deg)
        s = u_ref[0:NPAD, :] + u_ref[NPAD : 2 * NPAD, :] + t_ref[...]
        o_ref[...] = dinv * s + b_ref[...]

    return pl.pallas_call(
        body, out_shape=jax.ShapeDtypeStruct((NPAD, 16), jnp.float32)
    )(u2, t2, degp, b2p)


def kernel(x, edge_index, W1, b1, W2, b2):
    N, D = x.shape
    E = edge_index.shape[1]
    H = W1.shape[1]
    NPAD = -(-N // 512) * 512

    B = 80
    src80 = edge_index[0].reshape(E // B, B)
    dst80 = edge_index[1].reshape(E // B, B)
    xp = jnp.pad(x, ((0, NPAD - N), (0, 0)))
    w1p = jnp.pad(W1, ((0, 0), (0, 112 - H)))
    b1p = jnp.pad(b1, (0, 112 - H)).reshape(1, 112)
    w2p = jnp.pad(W2, ((0, 112 - H), (0, 15)))
    b2p = jnp.broadcast_to(b2.reshape(1, 1), (1, 16))

    seg16 = _make_segsum(16, NPAD, E, B)
    seg112 = _make_segsum_sb(112, NPAD, E, B)
    deghist = _make_deghist(NPAD, E, B)

    degp = deghist(dst80)
    hp = _tc1(xp, w1p, degp, NPAD)
    u1 = seg112(hp, src80, dst80)
    t2 = _tc2(u1, hp, degp, b1p, w2p, NPAD)
    u2 = seg16(t2, src80, dst80)
    o16 = _tc3(u2, t2, degp, b2p, NPAD)
    return o16[:N, 0:1]


# speedup vs baseline: 44.1535x; 1.0597x over previous
import functools

import jax
import jax.numpy as jnp
from jax import lax
from jax.experimental import pallas as pl
from jax.experimental.pallas import tpu as pltpu
from jax.experimental.pallas import tpu_sc as plsc

_NW = 32
_NS = 5


def _zero_rows(rows_ref, s, nrow, wc):
    zero16 = jnp.zeros((16,), jnp.float32)

    def zrow(r, carry):
        for j in range(wc):
            rows_ref[s, r, pl.ds(j * 16, 16)] = zero16
        return carry

    lax.fori_loop(0, nrow, zrow, 0)


def _zero_acc(rows_ref, acc_sh, sid, rpt, sem, B):
    nz = rpt // B
    cps = [
        pltpu.make_async_copy(
            rows_ref.at[0], acc_sh.at[pl.ds(sid * rpt + k * B, B)], sem
        )
        for k in range(nz)
    ]
    for cp in cps:
        cp.start()
    for cp in cps:
        cp.wait()


def _make_segsum(W, NPAD, E, B):
    EPW = E // _NW
    NCH = EPW // B
    NH = NCH // _NS
    RPT = NPAD // 16
    assert E % _NW == 0 and EPW % B == 0 and NCH % _NS == 0 and RPT % B == 0
    WC = W // 16
    assert W % 16 == 0 and (W * 4) % 64 == 0

    mesh = plsc.VectorSubcoreMesh(core_axis_name="c", subcore_axis_name="s")

    @functools.partial(
        pl.kernel,
        mesh=mesh,
        compiler_params=pltpu.CompilerParams(use_tc_tiling_on_sc=False),
        out_type=jax.ShapeDtypeStruct((2 * NPAD, W), jnp.float32),
        scratch_types=[
            pltpu.VMEM((NCH, B), jnp.int32),
            pltpu.VMEM((NCH, B), jnp.int32),
            pltpu.VMEM((_NS, B, W), jnp.float32),
            pltpu.VMEM_SHARED((NPAD, W), jnp.float32),
            pltpu.SemaphoreType.DMA((_NS,)),
            pltpu.SemaphoreType.DMA((_NS,)),
            pltpu.SemaphoreType.DMA,
        ],
    )
    def seg(table_hbm, ei_hbm, out_hbm, src_l, dst_l, rows_v,
            acc_sh, gsem, ssem, zsem):
        cid = lax.axis_index("c")
        sid = lax.axis_index("s")
        wid = cid * 16 + sid

        pltpu.make_async_copy(ei_hbm.at[0, pl.ds(wid * NCH, NCH)], src_l, zsem).start()
        pltpu.make_async_copy(ei_hbm.at[1, pl.ds(wid * NCH, NCH)], dst_l, zsem).start()
        _zero_rows(rows_v, 0, B, WC)
        pltpu.make_async_copy(ei_hbm.at[0, pl.ds(wid * NCH, NCH)], src_l, zsem).wait()
        pltpu.make_async_copy(ei_hbm.at[1, pl.ds(wid * NCH, NCH)], dst_l, zsem).wait()
        _zero_acc(rows_v, acc_sh, sid, RPT, zsem, B)
        plsc.subcore_barrier()

        def gather(g, s):
            return pltpu.make_async_copy(
                table_hbm.at[src_l.at[g]], rows_v.at[s], gsem.at[s]
            )

        def scatter(g, s):
            return pltpu.make_async_copy(
                rows_v.at[s], acc_sh.at[dst_l.at[g]], ssem.at[s]
            )

        for s in range(_NS):
            gather(s, s).start()

        def hyper(h, carry):
            g0 = h * _NS
            for s in range(_NS):
                gather(g0 + s, s).wait()
                scatter(g0 + s, s).start(add=True)
            for s in range(_NS):
                scatter(g0 + s, s).wait()
                gather(g0 + _NS + s, s).start()
            return carry

        lax.fori_loop(0, NH - 1, hyper, 0)

        g0 = (NH - 1) * _NS
        for s in range(_NS):
            gather(g0 + s, s).wait()
            scatter(g0 + s, s).start(add=True)
        for s in range(_NS):
            scatter(g0 + s, s).wait()

        plsc.subcore_barrier()
        pltpu.sync_copy(
            acc_sh.at[pl.ds(sid * RPT, RPT)],
            out_hbm.at[pl.ds(cid * NPAD + sid * RPT, RPT)],
        )

    return seg


def _make_segsum_sb(W, NPAD, E, B):
    EPW = E // _NW
    NCH = EPW // B
    SB = 5 * _NS
    NSB = NCH // SB
    RPT = NPAD // 16
    assert E % _NW == 0 and EPW % B == 0 and NCH % SB == 0 and RPT % B == 0
    WC = W // 16
    assert W % 16 == 0 and (W * 4) % 64 == 0

    mesh = plsc.VectorSubcoreMesh(core_axis_name="c", subcore_axis_name="s")

    @functools.partial(
        pl.kernel,
        mesh=mesh,
        compiler_params=pltpu.CompilerParams(use_tc_tiling_on_sc=False),
        out_type=jax.ShapeDtypeStruct((2 * NPAD, W), jnp.float32),
        scratch_types=[
            pltpu.VMEM((2 * SB, B), jnp.int32),
            pltpu.VMEM((2 * SB, B), jnp.int32),
            pltpu.VMEM((_NS, B, W), jnp.float32),
            pltpu.VMEM_SHARED((NPAD, W), jnp.float32),
            pltpu.SemaphoreType.DMA((_NS,)),
            pltpu.SemaphoreType.DMA((_NS,)),
            pltpu.SemaphoreType.DMA,
        ],
    )
    def seg(table_hbm, ei_hbm, out_hbm, src_l, dst_l, rows_v,
            acc_sh, gsem, ssem, zsem):
        cid = lax.axis_index("c")
        sid = lax.axis_index("s")
        wid = cid * 16 + sid
        base = wid * NCH

        def stage(sb, buf):
            return (
                pltpu.make_async_copy(
                    ei_hbm.at[0, pl.ds(base + sb * SB, SB)],
                    src_l.at[pl.ds(buf * SB, SB)], zsem),
                pltpu.make_async_copy(
                    ei_hbm.at[1, pl.ds(base + sb * SB, SB)],
                    dst_l.at[pl.ds(buf * SB, SB)], zsem),
            )

        for cp in stage(0, 0):
            cp.start()
        _zero_rows(rows_v, 0, B, WC)
        for cp in stage(0, 0):
            cp.wait()
        _zero_acc(rows_v, acc_sh, sid, RPT, zsem, B)
        plsc.subcore_barrier()

        def gather(buf, q, s):
            return pltpu.make_async_copy(
                table_hbm.at[src_l.at[buf * SB + q]], rows_v.at[s], gsem.at[s]
            )

        def scatter(buf, q, s):
            return pltpu.make_async_copy(
                rows_v.at[s], acc_sh.at[dst_l.at[buf * SB + q]], ssem.at[s]
            )

        for s in range(_NS):
            gather(0, s, s).start()

        def sbloop(sb, carry):
            b = sb % 2
            last = sb == NSB - 1

            @pl.when(jnp.logical_not(last))
            def _():
                for cp in stage(sb + 1, 1 - b):
                    cp.start()

            def hyper(h, c2):
                q0 = h * _NS
                for s in range(_NS):
                    gather(b, q0 + s, s).wait()
                    scatter(b, q0 + s, s).start(add=True)
                for s in range(_NS):
                    scatter(b, q0 + s, s).wait()
                    gather(b, q0 + _NS + s, s).start()
                return c2

            lax.fori_loop(0, SB // _NS - 1, hyper, 0)
            q0 = SB - _NS
            for s in range(_NS):
                gather(b, q0 + s, s).wait()
                scatter(b, q0 + s, s).start(add=True)
            for s in range(_NS):
                scatter(b, q0 + s, s).wait()

            @pl.when(jnp.logical_not(last))
            def _():
                for cp in stage(sb + 1, 1 - b):
                    cp.wait()
                for s in range(_NS):
                    gather(1 - b, s, s).start()

            return carry

        lax.fori_loop(0, NSB, sbloop, 0)

        plsc.subcore_barrier()
        pltpu.sync_copy(
            acc_sh.at[pl.ds(sid * RPT, RPT)],
            out_hbm.at[pl.ds(cid * NPAD + sid * RPT, RPT)],
        )

    return seg


def _make_deghist(NPAD, E, B):
    EPW = E // _NW
    NCH = EPW // B
    RPT = NPAD // 16
    assert E % _NW == 0 and EPW % B == 0 and RPT % B == 0
    W = 16

    mesh = plsc.VectorSubcoreMesh(core_axis_name="c", subcore_axis_name="s")

    @functools.partial(
        pl.kernel,
        mesh=mesh,
        compiler_params=pltpu.CompilerParams(use_tc_tiling_on_sc=False),
        out_type=jax.ShapeDtypeStruct((2 * NPAD, W), jnp.float32),
        scratch_types=[
            pltpu.VMEM((NCH, B), jnp.int32),
            pltpu.VMEM((_NS, B, W), jnp.float32),
            pltpu.VMEM_SHARED((NPAD, W), jnp.float32),
            pltpu.SemaphoreType.DMA((_NS,)),
            pltpu.SemaphoreType.DMA,
        ],
    )
    def deg(ei_hbm, out_hbm, dst_l, rows_v, acc_sh, ssem, zsem):
        cid = lax.axis_index("c")
        sid = lax.axis_index("s")
        wid = cid * 16 + sid

        pltpu.make_async_copy(ei_hbm.at[1, pl.ds(wid * NCH, NCH)], dst_l, zsem).start()
        _zero_rows(rows_v, 0, B, 1)
        one16 = jnp.where(lax.iota(jnp.int32, 16) == 0, 1.0, 0.0).astype(jnp.float32)

        def orow(r, carry):
            rows_v[0, r, pl.ds(0, 16)] = one16
            return carry

        pltpu.make_async_copy(ei_hbm.at[1, pl.ds(wid * NCH, NCH)], dst_l, zsem).wait()
        _zero_acc(rows_v, acc_sh, sid, RPT, zsem, B)
        lax.fori_loop(0, B, orow, 0)
        plsc.subcore_barrier()

        def scatter(g, s):
            return pltpu.make_async_copy(
                rows_v.at[0], acc_sh.at[dst_l.at[g]], ssem.at[s]
            )

        for s in range(_NS):
            scatter(s, s).start(add=True)

        def hyper(h, carry):
            g0 = h * _NS
            for s in range(_NS):
                scatter(g0 + s, s).wait()
                scatter(g0 + _NS + s, s).start(add=True)
            return carry

        lax.fori_loop(0, NCH // _NS - 1, hyper, 0)
        g0 = NCH - _NS
        for s in range(_NS):
            scatter(g0 + s, s).wait()

        plsc.subcore_barrier()
        pltpu.sync_copy(
            acc_sh.at[pl.ds(sid * RPT, RPT)],
            out_hbm.at[pl.ds(cid * NPAD + sid * RPT, RPT)],
        )

    return deg


def _tc_matmul(x, w1p, N, NPAD):
    def body(x_ref, w_ref, o_ref):
        o_ref[0:N, :] = jnp.dot(x_ref[...], w_ref[...],
                                preferred_element_type=jnp.float32,
                                precision=lax.Precision.HIGHEST)
        o_ref[N:NPAD, :] = jnp.zeros((NPAD - N, 112), jnp.float32)

    return pl.pallas_call(
        body, out_shape=jax.ShapeDtypeStruct((NPAD, 112), jnp.float32)
    )(x, w1p)


def _tc_scale(h, degp, NPAD):
    def body(h_ref, d_ref, o_ref):
        deg = d_ref[0:NPAD, 0:1] + d_ref[NPAD : 2 * NPAD, 0:1] + 1.0
        o_ref[...] = h_ref[...] * lax.rsqrt(deg)

    return pl.pallas_call(
        body, out_shape=jax.ShapeDtypeStruct((NPAD, 112), jnp.float32)
    )(h, degp)


def _tc2(u1, hp, degp, b1p, w2p, NPAD):
    def body(u_ref, h_ref, d_ref, b_ref, w_ref, o_ref):
        deg = d_ref[0:NPAD, 0:1] + d_ref[NPAD : 2 * NPAD, 0:1] + 1.0
        dinv = lax.rsqrt(deg)
        s = u_ref[0:NPAD, :] + u_ref[NPAD : 2 * NPAD, :] + h_ref[...]
        r = jnp.maximum(dinv * s + b_ref[...], 0.0)
        h2 = jnp.dot(r, w_ref[...],
                     preferred_element_type=jnp.float32,
                     precision=lax.Precision.HIGHEST)
        o_ref[...] = h2 * dinv

    return pl.pallas_call(
        body, out_shape=jax.ShapeDtypeStruct((NPAD, 16), jnp.float32)
    )(u1, hp, degp, b1p, w2p)


def _tc3(u2, t2, degp, b2p, NPAD):
    def body(u_ref, t_ref, d_ref, b_ref, o_ref):
        deg = d_ref[0:NPAD, 0:1] + d_ref[NPAD : 2 * NPAD, 0:1] + 1.0
        dinv = lax.rsqrt(deg)
        s = u_ref[0:NPAD, :] + u_ref[NPAD : 2 * NPAD, :] + t_ref[...]
        o_ref[...] = dinv * s + b_ref[...]

    return pl.pallas_call(
        body, out_shape=jax.ShapeDtypeStruct((NPAD, 16), jnp.float32)
    )(u2, t2, degp, b2p)


def kernel(x, edge_index, W1, b1, W2, b2):
    N, D = x.shape
    E = edge_index.shape[1]
    H = W1.shape[1]
    NPAD = -(-N // 512) * 512

    B = 80
    ei3 = edge_index.reshape(2, E // B, B)
    w1p = jnp.pad(W1, ((0, 0), (0, 112 - H)))
    b1p = jnp.pad(b1, (0, 112 - H)).reshape(1, 112)
    w2p = jnp.pad(W2, ((0, 112 - H), (0, 15)))
    b2p = jnp.broadcast_to(b2.reshape(1, 1), (1, 16))

    seg16 = _make_segsum(16, NPAD, E, B)
    seg112 = _make_segsum_sb(112, NPAD, E, B)
    deghist = _make_deghist(NPAD, E, B)

    degp = deghist(ei3)
    h = _tc_matmul(x, w1p, N, NPAD)
    hp = _tc_scale(h, degp, NPAD)
    u1 = seg112(hp, ei3)
    t2 = _tc2(u1, hp, degp, b1p, w2p, NPAD)
    u2 = seg16(t2, ei3)
    o16 = _tc3(u2, t2, degp, b2p, NPAD)
    return o16[:N, 0:1]
